# Initial kernel scaffold; baseline (speedup 1.0000x reference)
import jax
import jax.numpy as jnp
from jax.experimental import pallas as pl


def kernel(x, edges, W1, b1, W2, b2, Wl, bl):
    raise NotImplementedError("write your pallas kernel here")


# speedup vs baseline: 16.6304x; 16.6304x over previous
import functools

import jax
import jax.numpy as jnp
from jax import lax
from jax.experimental import pallas as pl
from jax.experimental.pallas import tpu as pltpu
from jax.experimental.pallas import tpu_sc as plsc

NN = 100000
EE = 3200000
HH = 16
NPAD = 100352

NC = 2
NS = 16
NW = NC * NS
EPT = EE // NW
CH = 80
NCHUNK = EPT // CH
SP = NPAD // NS
ZB = 1568
RZ = 224

_mesh = plsc.VectorSubcoreMesh(core_axis_name="c", subcore_axis_name="s")

_f32 = jnp.float32
_i32 = jnp.int32


def _fill(ref, n, vec16):
    for i in range(n // 16):
        ref[pl.ds(i * 16, 16)] = vec16


@functools.partial(
    pl.kernel,
    out_type=[jax.ShapeDtypeStruct((NPAD,), _f32)] * 2,
    mesh=_mesh,
    scratch_types=[
        pltpu.VMEM((CH,), _i32),
        pltpu.VMEM((CH,), _f32),
        pltpu.VMEM((ZB,), _f32),
        pltpu.VMEM_SHARED((NPAD,), _f32),
    ],
)
def _sc_deg(dst_hbm, out0_hbm, out1_hbm, idx_d, ones_v, zb_v, acc):
    c = lax.axis_index("c")
    s = lax.axis_index("s")
    _fill(ones_v, CH, jnp.ones((16,), _f32))
    _fill(zb_v, ZB, jnp.zeros((16,), _f32))
    for j in range(SP // ZB):
        pltpu.sync_copy(zb_v, acc.at[pl.ds(s * SP + j * ZB, ZB)])
    plsc.subcore_barrier()
    base = (c * NS + s) * EPT

    def body(k, carry):
        off = base + k * CH
        pltpu.sync_copy(dst_hbm.at[pl.ds(off, CH)], idx_d)
        pltpu.sync_copy(ones_v, acc.at[idx_d], add=True)
        return carry

    lax.fori_loop(0, NCHUNK, body, 0)
    plsc.subcore_barrier()

    @pl.when(c == 0)
    def _():
        pltpu.sync_copy(acc.at[pl.ds(s * SP, SP)], out0_hbm.at[pl.ds(s * SP, SP)])

    @pl.when(c == 1)
    def _():
        pltpu.sync_copy(acc.at[pl.ds(s * SP, SP)], out1_hbm.at[pl.ds(s * SP, SP)])


@functools.partial(
    pl.kernel,
    out_type=[jax.ShapeDtypeStruct((NPAD,), _f32)] * 2,
    mesh=_mesh,
    scratch_types=[
        pltpu.VMEM((CH,), _i32),
        pltpu.VMEM((CH,), _i32),
        pltpu.VMEM((CH,), _f32),
        pltpu.VMEM((ZB,), _f32),
        pltpu.VMEM_SHARED((NPAD,), _f32),
        pltpu.SemaphoreType.DMA,
    ],
)
def _sc_agg1(src_hbm, dst_hbm, p1_hbm, out0_hbm, out1_hbm,
             idx_s, idx_d, vals_v, zb_v, acc, sem):
    c = lax.axis_index("c")
    s = lax.axis_index("s")
    _fill(zb_v, ZB, jnp.zeros((16,), _f32))
    for j in range(SP // ZB):
        pltpu.sync_copy(zb_v, acc.at[pl.ds(s * SP + j * ZB, ZB)])
    plsc.subcore_barrier()
    base = (c * NS + s) * EPT

    def body(k, carry):
        off = base + k * CH
        pltpu.sync_copy(src_hbm.at[pl.ds(off, CH)], idx_s)
        pltpu.sync_copy(dst_hbm.at[pl.ds(off, CH)], idx_d)
        pltpu.async_copy(p1_hbm.at[idx_s], vals_v, sem).wait()
        pltpu.sync_copy(vals_v, acc.at[idx_d], add=True)
        return carry

    lax.fori_loop(0, NCHUNK, body, 0)
    plsc.subcore_barrier()

    @pl.when(c == 0)
    def _():
        pltpu.sync_copy(acc.at[pl.ds(s * SP, SP)], out0_hbm.at[pl.ds(s * SP, SP)])

    @pl.when(c == 1)
    def _():
        pltpu.sync_copy(acc.at[pl.ds(s * SP, SP)], out1_hbm.at[pl.ds(s * SP, SP)])


@functools.partial(
    pl.kernel,
    out_type=[jax.ShapeDtypeStruct((NPAD, HH), _f32)] * 2,
    mesh=_mesh,
    compiler_params=pltpu.CompilerParams(use_tc_tiling_on_sc=False),
    scratch_types=[
        pltpu.VMEM((CH,), _i32),
        pltpu.VMEM((CH,), _i32),
        pltpu.VMEM((CH, HH), _f32),
        pltpu.VMEM((RZ, HH), _f32),
        pltpu.VMEM_SHARED((NPAD, HH), _f32),
        pltpu.SemaphoreType.DMA,
    ],
)
def _sc_agg2(src_hbm, dst_hbm, p2_hbm, out0_hbm, out1_hbm,
             idx_s, idx_d, rows_v, zr_v, acc, sem):
    c = lax.axis_index("c")
    s = lax.axis_index("s")
    z16 = jnp.zeros((16,), _f32)
    for i in range(RZ):
        zr_v[i, :] = z16
    for j in range(SP // RZ):
        pltpu.sync_copy(zr_v, acc.at[pl.ds(s * SP + j * RZ, RZ), :])
    plsc.subcore_barrier()
    base = (c * NS + s) * EPT

    def body(k, carry):
        off = base + k * CH
        pltpu.sync_copy(src_hbm.at[pl.ds(off, CH)], idx_s)
        pltpu.sync_copy(dst_hbm.at[pl.ds(off, CH)], idx_d)
        pltpu.async_copy(p2_hbm.at[idx_s], rows_v, sem).wait()
        pltpu.sync_copy(rows_v, acc.at[idx_d], add=True)
        return carry

    lax.fori_loop(0, NCHUNK, body, 0)
    plsc.subcore_barrier()

    @pl.when(c == 0)
    def _():
        pltpu.sync_copy(acc.at[pl.ds(s * SP, SP), :],
                        out0_hbm.at[pl.ds(s * SP, SP), :])

    @pl.when(c == 1)
    def _():
        pltpu.sync_copy(acc.at[pl.ds(s * SP, SP), :],
                        out1_hbm.at[pl.ds(s * SP, SP), :])


def _stage0(dega, degb, xp):
    def body(da_r, db_r, x_r, dinv_r, p1_r):
        deg = da_r[:] + db_r[:] + 1.0
        di = lax.rsqrt(deg)
        dinv_r[:] = di
        p1_r[:] = di * x_r[:]

    return pl.pallas_call(
        body,
        out_shape=[jax.ShapeDtypeStruct((NPAD,), _f32),
                   jax.ShapeDtypeStruct((NPAD,), _f32)],
    )(dega, degb, xp)


GB = 49
BR = NPAD // GB


def _stage1(a1a, a1b, dinv, xp, W1, b1, W2):
    def body(aa_r, ab_r, dinv_r, x_r, W1_r, b1_r, W2_r, p2_r):
        di = dinv_r[:]
        s1 = di * (aa_r[:] + ab_r[:]) + di * di * x_r[:]
        h1 = jnp.maximum(s1[:, None] * W1_r[:, :] + b1_r[:, :], 0.0)
        p2_r[:, :] = di[:, None] * jnp.dot(
            h1, W2_r[:, :], preferred_element_type=_f32)

    return pl.pallas_call(
        body,
        grid=(GB,),
        in_specs=[
            pl.BlockSpec((BR,), lambda i: (i,)),
            pl.BlockSpec((BR,), lambda i: (i,)),
            pl.BlockSpec((BR,), lambda i: (i,)),
            pl.BlockSpec((BR,), lambda i: (i,)),
            pl.BlockSpec((1, HH), lambda i: (0, 0)),
            pl.BlockSpec((1, HH), lambda i: (0, 0)),
            pl.BlockSpec((HH, HH), lambda i: (0, 0)),
        ],
        out_specs=pl.BlockSpec((BR, HH), lambda i: (i, 0)),
        out_shape=jax.ShapeDtypeStruct((NPAD, HH), _f32),
    )(a1a, a1b, dinv, xp, W1, b1, W2)


def _stage2(a2a, a2b, p2, dinv, b2, Wl, bl):
    def body(aa_r, ab_r, p2_r, dinv_r, b2_r, Wl_r, bl_r, o_r):
        t = aa_r[:, :] + ab_r[:, :] + p2_r[:, :]
        pre = dinv_r[:][:, None] * t + b2_r[:, :]
        h2 = jnp.maximum(pre, 0.0)
        o_r[:, :] = jnp.dot(h2, Wl_r[:, :],
                            preferred_element_type=_f32) + bl_r[:, :]

    return pl.pallas_call(
        body,
        grid=(GB,),
        in_specs=[
            pl.BlockSpec((BR, HH), lambda i: (i, 0)),
            pl.BlockSpec((BR, HH), lambda i: (i, 0)),
            pl.BlockSpec((BR, HH), lambda i: (i, 0)),
            pl.BlockSpec((BR,), lambda i: (i,)),
            pl.BlockSpec((1, HH), lambda i: (0, 0)),
            pl.BlockSpec((HH, 4), lambda i: (0, 0)),
            pl.BlockSpec((1, 4), lambda i: (0, 0)),
        ],
        out_specs=pl.BlockSpec((BR, 4), lambda i: (i, 0)),
        out_shape=jax.ShapeDtypeStruct((NPAD, 4), _f32),
    )(a2a, a2b, p2, dinv, b2, Wl, bl)


def kernel(x, edges, W1, b1, W2, b2, Wl, bl):
    xp = jnp.pad(x[:, 0], (0, NPAD - NN))
    src1d = edges[0]
    dst1d = edges[1]
    dega, degb = _sc_deg(dst1d)
    dinv, p1 = _stage0(dega, degb, xp)
    a1a, a1b = _sc_agg1(src1d, dst1d, p1)
    p2 = _stage1(a1a, a1b, dinv, xp, W1, b1.reshape(1, HH), W2)
    a2a, a2b = _sc_agg2(src1d, dst1d, p2)
    outp = _stage2(a2a, a2b, p2, dinv, b2.reshape(1, HH), Wl,
                   bl.reshape(1, 4))
    return outp[:NN]


# speedup vs baseline: 73.0906x; 4.3950x over previous
import functools

import jax
import jax.numpy as jnp
from jax import lax
from jax.experimental import pallas as pl
from jax.experimental.pallas import tpu as pltpu
from jax.experimental.pallas import tpu_sc as plsc

NN = 100000
EE = 3200000
HH = 16
NPAD = 100352

NC = 2
NS = 16
NW = NC * NS
EPT = EE // NW
CH = 80
NCHUNK = EPT // CH
SP = NPAD // NS
ZB = 1568
RZ = 224

NBUF = 8
D1 = 3
D2 = 6

_mesh = plsc.VectorSubcoreMesh(core_axis_name="c", subcore_axis_name="s")
_noscale = pltpu.CompilerParams(use_tc_tiling_on_sc=False)

_f32 = jnp.float32
_i32 = jnp.int32


def _fill(ref, n, vec16):
    for i in range(n // 16):
        ref[pl.ds(i * 16, 16)] = vec16


def _zero_acc_1d(acc, zb_v, s):
    _fill(zb_v, ZB, jnp.zeros((16,), _f32))
    for j in range(SP // ZB):
        pltpu.sync_copy(zb_v, acc.at[pl.ds(s * SP + j * ZB, ZB)])


def _flush_1d(acc, out0_hbm, out1_hbm, c, s):
    @pl.when(c == 0)
    def _():
        pltpu.sync_copy(acc.at[pl.ds(s * SP, SP)], out0_hbm.at[pl.ds(s * SP, SP)])

    @pl.when(c == 1)
    def _():
        pltpu.sync_copy(acc.at[pl.ds(s * SP, SP)], out1_hbm.at[pl.ds(s * SP, SP)])


@functools.partial(
    pl.kernel,
    out_type=[jax.ShapeDtypeStruct((NPAD,), _f32)] * 2,
    mesh=_mesh,
    scratch_types=[
        pltpu.VMEM((NBUF, CH), _i32),
        pltpu.VMEM((CH,), _f32),
        pltpu.VMEM((ZB,), _f32),
        pltpu.VMEM_SHARED((NPAD,), _f32),
        pltpu.SemaphoreType.DMA((NBUF,)),
        pltpu.SemaphoreType.DMA((NBUF,)),
    ],
)
def _sc_deg(dst_hbm, out0_hbm, out1_hbm, idx_d, ones_v, zb_v, acc, semE, semS):
    c = lax.axis_index("c")
    s = lax.axis_index("s")
    _fill(ones_v, CH, jnp.ones((16,), _f32))
    _zero_acc_1d(acc, zb_v, s)
    plsc.subcore_barrier()
    base = (c * NS + s) * EPT
    steps = NCHUNK + D1
    groups = (steps + NBUF - 1) // NBUF

    def group(g, carry):
        for b in range(NBUF):
            k0 = g * NBUF + b
            ks = k0 - D1
            bs = (b - D1) % NBUF

            @pl.when((ks >= 0) & (ks < NCHUNK))
            def _():
                pltpu.make_async_copy(
                    dst_hbm.at[pl.ds(0, CH)], idx_d.at[bs], semE.at[bs]).wait()
                pltpu.async_copy(
                    ones_v, acc.at[idx_d.at[bs]], semS.at[bs], add=True)

            @pl.when(k0 < NCHUNK)
            def _():
                @pl.when(k0 >= NBUF)
                def _():
                    pltpu.make_async_copy(
                        ones_v, acc.at[idx_d.at[b]], semS.at[b]).wait()

                off = base + k0 * CH
                pltpu.async_copy(
                    dst_hbm.at[pl.ds(off, CH)], idx_d.at[b], semE.at[b])

        return carry

    lax.fori_loop(0, groups, group, 0)
    for b in range(NBUF):
        last = NCHUNK - NBUF + b

        @pl.when((last >= 0) & (last < NCHUNK))
        def _():
            pltpu.make_async_copy(ones_v, acc.at[idx_d.at[b]], semS.at[b]).wait()

    plsc.subcore_barrier()
    _flush_1d(acc, out0_hbm, out1_hbm, c, s)


@functools.partial(
    pl.kernel,
    out_type=[jax.ShapeDtypeStruct((NPAD,), _f32)] * 2,
    mesh=_mesh,
    compiler_params=_noscale,
    scratch_types=[
        pltpu.VMEM((NBUF, CH), _i32),
        pltpu.VMEM((NBUF, CH), _i32),
        pltpu.VMEM((NBUF, CH), _f32),
        pltpu.VMEM((ZB,), _f32),
        pltpu.VMEM_SHARED((NPAD,), _f32),
        pltpu.SemaphoreType.DMA((NBUF,)),
        pltpu.SemaphoreType.DMA((NBUF,)),
        pltpu.SemaphoreType.DMA((NBUF,)),
    ],
)
def _sc_agg1(src_hbm, dst_hbm, p1_hbm, out0_hbm, out1_hbm,
             idx_s, idx_d, vals, zb_v, acc, semE, semG, semS):
    c = lax.axis_index("c")
    s = lax.axis_index("s")
    _zero_acc_1d(acc, zb_v, s)
    plsc.subcore_barrier()
    base = (c * NS + s) * EPT
    steps = NCHUNK + D2
    groups = (steps + NBUF - 1) // NBUF

    def group(g, carry):
        for b in range(NBUF):
            k0 = g * NBUF + b

            ks = k0 - D2
            bs = (b - D2) % NBUF

            @pl.when((ks >= 0) & (ks < NCHUNK))
            def _():
                pltpu.make_async_copy(
                    p1_hbm.at[idx_s.at[bs]], vals.at[bs], semG.at[bs]).wait()
                pltpu.async_copy(
                    vals.at[bs], acc.at[idx_d.at[bs]], semS.at[bs], add=True)

            kg = k0 - D1
            bg = (b - D1) % NBUF

            @pl.when((kg >= 0) & (kg < NCHUNK))
            def _():
                pltpu.make_async_copy(
                    src_hbm.at[pl.ds(0, CH)], idx_s.at[bg], semE.at[bg]).wait()
                pltpu.make_async_copy(
                    dst_hbm.at[pl.ds(0, CH)], idx_d.at[bg], semE.at[bg]).wait()
                pltpu.async_copy(
                    p1_hbm.at[idx_s.at[bg]], vals.at[bg], semG.at[bg])

            @pl.when(k0 < NCHUNK)
            def _():
                @pl.when(k0 >= NBUF)
                def _():
                    pltpu.make_async_copy(
                        vals.at[b], acc.at[idx_d.at[b]], semS.at[b]).wait()

                off = base + k0 * CH
                pltpu.async_copy(
                    src_hbm.at[pl.ds(off, CH)], idx_s.at[b], semE.at[b])
                pltpu.async_copy(
                    dst_hbm.at[pl.ds(off, CH)], idx_d.at[b], semE.at[b])

        return carry

    lax.fori_loop(0, groups, group, 0)
    for b in range(NBUF):
        last = NCHUNK - NBUF + b

        @pl.when((last >= 0) & (last < NCHUNK))
        def _():
            pltpu.make_async_copy(
                vals.at[b], acc.at[idx_d.at[b]], semS.at[b]).wait()

    plsc.subcore_barrier()
    _flush_1d(acc, out0_hbm, out1_hbm, c, s)


@functools.partial(
    pl.kernel,
    out_type=[jax.ShapeDtypeStruct((NPAD, HH), _f32)] * 2,
    mesh=_mesh,
    compiler_params=_noscale,
    scratch_types=[
        pltpu.VMEM((NBUF, CH), _i32),
        pltpu.VMEM((NBUF, CH), _i32),
        pltpu.VMEM((NBUF, CH, HH), _f32),
        pltpu.VMEM((RZ, HH), _f32),
        pltpu.VMEM_SHARED((NPAD, HH), _f32),
        pltpu.SemaphoreType.DMA((NBUF,)),
        pltpu.SemaphoreType.DMA((NBUF,)),
        pltpu.SemaphoreType.DMA((NBUF,)),
    ],
)
def _sc_agg2(src_hbm, dst_hbm, p2_hbm, out0_hbm, out1_hbm,
             idx_s, idx_d, rows, zr_v, acc, semE, semG, semS):
    c = lax.axis_index("c")
    s = lax.axis_index("s")
    z16 = jnp.zeros((16,), _f32)
    for i in range(RZ):
        zr_v[i, :] = z16
    for j in range(SP // RZ):
        pltpu.sync_copy(zr_v, acc.at[pl.ds(s * SP + j * RZ, RZ), :])
    plsc.subcore_barrier()
    base = (c * NS + s) * EPT
    steps = NCHUNK + D2
    groups = (steps + NBUF - 1) // NBUF

    def group(g, carry):
        for b in range(NBUF):
            k0 = g * NBUF + b

            ks = k0 - D2
            bs = (b - D2) % NBUF

            @pl.when((ks >= 0) & (ks < NCHUNK))
            def _():
                pltpu.make_async_copy(
                    p2_hbm.at[idx_s.at[bs]], rows.at[bs], semG.at[bs]).wait()
                pltpu.async_copy(
                    rows.at[bs], acc.at[idx_d.at[bs]], semS.at[bs], add=True)

            kg = k0 - D1
            bg = (b - D1) % NBUF

            @pl.when((kg >= 0) & (kg < NCHUNK))
            def _():
                pltpu.make_async_copy(
                    src_hbm.at[pl.ds(0, CH)], idx_s.at[bg], semE.at[bg]).wait()
                pltpu.make_async_copy(
                    dst_hbm.at[pl.ds(0, CH)], idx_d.at[bg], semE.at[bg]).wait()
                pltpu.async_copy(
                    p2_hbm.at[idx_s.at[bg]], rows.at[bg], semG.at[bg])

            @pl.when(k0 < NCHUNK)
            def _():
                @pl.when(k0 >= NBUF)
                def _():
                    pltpu.make_async_copy(
                        rows.at[b], acc.at[idx_d.at[b]], semS.at[b]).wait()

                off = base + k0 * CH
                pltpu.async_copy(
                    src_hbm.at[pl.ds(off, CH)], idx_s.at[b], semE.at[b])
                pltpu.async_copy(
                    dst_hbm.at[pl.ds(off, CH)], idx_d.at[b], semE.at[b])

        return carry

    lax.fori_loop(0, groups, group, 0)
    for b in range(NBUF):
        last = NCHUNK - NBUF + b

        @pl.when((last >= 0) & (last < NCHUNK))
        def _():
            pltpu.make_async_copy(
                rows.at[b], acc.at[idx_d.at[b]], semS.at[b]).wait()

    plsc.subcore_barrier()

    @pl.when(c == 0)
    def _():
        pltpu.sync_copy(acc.at[pl.ds(s * SP, SP), :],
                        out0_hbm.at[pl.ds(s * SP, SP), :])

    @pl.when(c == 1)
    def _():
        pltpu.sync_copy(acc.at[pl.ds(s * SP, SP), :],
                        out1_hbm.at[pl.ds(s * SP, SP), :])


def _stage0(dega, degb, xp):
    def body(da_r, db_r, x_r, dinv_r, p1_r):
        deg = da_r[:] + db_r[:] + 1.0
        di = lax.rsqrt(deg)
        dinv_r[:] = di
        p1_r[:] = di * x_r[:]

    return pl.pallas_call(
        body,
        out_shape=[jax.ShapeDtypeStruct((NPAD,), _f32),
                   jax.ShapeDtypeStruct((NPAD,), _f32)],
    )(dega, degb, xp)


GB = 49
BR = NPAD // GB


def _stage1(a1a, a1b, dinv, xp, W1, b1, W2):
    def body(aa_r, ab_r, dinv_r, x_r, W1_r, b1_r, W2_r, p2_r):
        di = dinv_r[:]
        s1 = di * (aa_r[:] + ab_r[:]) + di * di * x_r[:]
        h1 = jnp.maximum(s1[:, None] * W1_r[:, :] + b1_r[:, :], 0.0)
        p2_r[:, :] = di[:, None] * jnp.dot(
            h1, W2_r[:, :], preferred_element_type=_f32)

    return pl.pallas_call(
        body,
        grid=(GB,),
        in_specs=[
            pl.BlockSpec((BR,), lambda i: (i,)),
            pl.BlockSpec((BR,), lambda i: (i,)),
            pl.BlockSpec((BR,), lambda i: (i,)),
            pl.BlockSpec((BR,), lambda i: (i,)),
            pl.BlockSpec((1, HH), lambda i: (0, 0)),
            pl.BlockSpec((1, HH), lambda i: (0, 0)),
            pl.BlockSpec((HH, HH), lambda i: (0, 0)),
        ],
        out_specs=pl.BlockSpec((BR, HH), lambda i: (i, 0)),
        out_shape=jax.ShapeDtypeStruct((NPAD, HH), _f32),
    )(a1a, a1b, dinv, xp, W1, b1, W2)


def _stage2(a2a, a2b, p2, dinv, b2, Wl, bl):
    def body(aa_r, ab_r, p2_r, dinv_r, b2_r, Wl_r, bl_r, o_r):
        t = aa_r[:, :] + ab_r[:, :] + p2_r[:, :]
        pre = dinv_r[:][:, None] * t + b2_r[:, :]
        h2 = jnp.maximum(pre, 0.0)
        o_r[:, :] = jnp.dot(h2, Wl_r[:, :],
                            preferred_element_type=_f32) + bl_r[:, :]

    return pl.pallas_call(
        body,
        grid=(GB,),
        in_specs=[
            pl.BlockSpec((BR, HH), lambda i: (i, 0)),
            pl.BlockSpec((BR, HH), lambda i: (i, 0)),
            pl.BlockSpec((BR, HH), lambda i: (i, 0)),
            pl.BlockSpec((BR,), lambda i: (i,)),
            pl.BlockSpec((1, HH), lambda i: (0, 0)),
            pl.BlockSpec((HH, 4), lambda i: (0, 0)),
            pl.BlockSpec((1, 4), lambda i: (0, 0)),
        ],
        out_specs=pl.BlockSpec((BR, 4), lambda i: (i, 0)),
        out_shape=jax.ShapeDtypeStruct((NPAD, 4), _f32),
    )(a2a, a2b, p2, dinv, b2, Wl, bl)


def kernel(x, edges, W1, b1, W2, b2, Wl, bl):
    xp = jnp.pad(x[:, 0], (0, NPAD - NN))
    src1d = edges[0]
    dst1d = edges[1]
    dega, degb = _sc_deg(dst1d)
    dinv, p1 = _stage0(dega, degb, xp)
    a1a, a1b = _sc_agg1(src1d, dst1d, p1)
    p2 = _stage1(a1a, a1b, dinv, xp, W1, b1.reshape(1, HH), W2)
    a2a, a2b = _sc_agg2(src1d, dst1d, p2)
    outp = _stage2(a2a, a2b, p2, dinv, b2.reshape(1, HH), Wl,
                   bl.reshape(1, 4))
    return outp[:NN]


# speedup vs baseline: 100.6761x; 1.3774x over previous
import functools

import jax
import jax.numpy as jnp
from jax import lax
from jax.experimental import pallas as pl
from jax.experimental.pallas import tpu as pltpu
from jax.experimental.pallas import tpu_sc as plsc

NN = 100000
EE = 3200000
HH = 16
NPAD = 100352

NC = 2
NS = 16
NW = NC * NS
EPT = EE // NW
CH = 80
NCHUNK = EPT // CH
SP = NPAD // NS
ZB = 1568
RZ = 224

SS = 25
NSS = NCHUNK // SS
SSC = 5
NSSC = NCHUNK // SSC
NBUF = 3
D1 = 1
D2 = 2

_mesh = plsc.VectorSubcoreMesh(core_axis_name="c", subcore_axis_name="s")
_noscale = pltpu.CompilerParams(use_tc_tiling_on_sc=False)

_f32 = jnp.float32
_i32 = jnp.int32


def _fill(ref, n, vec16):
    for i in range(n // 16):
        ref[pl.ds(i * 16, 16)] = vec16


def _zero_acc_1d(acc, zb_v, s):
    _fill(zb_v, ZB, jnp.zeros((16,), _f32))
    for j in range(SP // ZB):
        pltpu.sync_copy(zb_v, acc.at[pl.ds(s * SP + j * ZB, ZB)])


def _flush_1d(acc, out0_hbm, out1_hbm, c, s):
    @pl.when(c == 0)
    def _():
        pltpu.sync_copy(acc.at[pl.ds(s * SP, SP)], out0_hbm.at[pl.ds(s * SP, SP)])

    @pl.when(c == 1)
    def _():
        pltpu.sync_copy(acc.at[pl.ds(s * SP, SP)], out1_hbm.at[pl.ds(s * SP, SP)])


@functools.partial(
    pl.kernel,
    out_type=[jax.ShapeDtypeStruct((NPAD,), _f32)] * 2,
    mesh=_mesh,
    compiler_params=_noscale,
    scratch_types=[
        pltpu.VMEM((NBUF, SS, CH), _i32),
        pltpu.VMEM((CH,), _f32),
        pltpu.VMEM((ZB,), _f32),
        pltpu.VMEM_SHARED((NPAD,), _f32),
        pltpu.SemaphoreType.DMA((NBUF,)),
        pltpu.SemaphoreType.DMA((NBUF,)),
    ],
)
def _sc_deg(dst2d_hbm, out0_hbm, out1_hbm, idx_d, ones_v, zb_v, acc,
            semE, semS):
    c = lax.axis_index("c")
    s = lax.axis_index("s")
    _fill(ones_v, CH, jnp.ones((16,), _f32))
    _zero_acc_1d(acc, zb_v, s)
    plsc.subcore_barrier()
    rbase = (c * NS + s) * NCHUNK
    steps = NSS + D1
    groups = (steps + NBUF - 1) // NBUF

    def group(g, carry):
        for b in range(NBUF):
            k0 = g * NBUF + b
            ks = k0 - D1
            bs = (b - D1) % NBUF

            @pl.when((ks >= 0) & (ks < NSS))
            def _():
                pltpu.make_async_copy(
                    dst2d_hbm.at[pl.ds(0, SS), :], idx_d.at[bs],
                    semE.at[bs]).wait()
                for i in range(SS):
                    pltpu.async_copy(
                        ones_v, acc.at[idx_d.at[bs, i]], semS.at[bs],
                        add=True)

            @pl.when(k0 < NSS)
            def _():
                @pl.when(k0 >= NBUF)
                def _():
                    pltpu.make_async_copy(
                        dst2d_hbm.at[pl.ds(0, SS), :], idx_d.at[b],
                        semS.at[b]).wait()

                row = rbase + k0 * SS
                pltpu.async_copy(
                    dst2d_hbm.at[pl.ds(row, SS), :], idx_d.at[b], semE.at[b])

        return carry

    lax.fori_loop(0, groups, group, 0)
    for b in range(NBUF):
        last = NSS - NBUF + b

        @pl.when((last >= 0) & (last < NSS))
        def _():
            pltpu.make_async_copy(
                dst2d_hbm.at[pl.ds(0, SS), :], idx_d.at[b], semS.at[b]).wait()

    plsc.subcore_barrier()
    _flush_1d(acc, out0_hbm, out1_hbm, c, s)


@functools.partial(
    pl.kernel,
    out_type=[jax.ShapeDtypeStruct((NPAD,), _f32)] * 2,
    mesh=_mesh,
    compiler_params=_noscale,
    scratch_types=[
        pltpu.VMEM((NBUF, SS, CH), _i32),
        pltpu.VMEM((NBUF, SS, CH), _i32),
        pltpu.VMEM((NBUF, SS * CH), _f32),
        pltpu.VMEM((ZB,), _f32),
        pltpu.VMEM_SHARED((NPAD,), _f32),
        pltpu.SemaphoreType.DMA((NBUF,)),
        pltpu.SemaphoreType.DMA((NBUF,)),
        pltpu.SemaphoreType.DMA((NBUF,)),
    ],
)
def _sc_agg1(src2d_hbm, dst2d_hbm, p1_hbm, out0_hbm, out1_hbm,
             idx_s, idx_d, vals, zb_v, acc, semE, semG, semS):
    c = lax.axis_index("c")
    s = lax.axis_index("s")
    _zero_acc_1d(acc, zb_v, s)
    plsc.subcore_barrier()
    rbase = (c * NS + s) * NCHUNK
    steps = NSS + D2
    groups = (steps + NBUF - 1) // NBUF

    def group(g, carry):
        for b in range(NBUF):
            k0 = g * NBUF + b

            ks = k0 - D2
            bs = (b - D2) % NBUF

            @pl.when((ks >= 0) & (ks < NSS))
            def _():
                pltpu.make_async_copy(
                    p1_hbm.at[pl.ds(0, SS * CH)], vals.at[bs],
                    semG.at[bs]).wait()
                for i in range(SS):
                    pltpu.async_copy(
                        vals.at[bs, pl.ds(i * CH, CH)],
                        acc.at[idx_d.at[bs, i]], semS.at[bs], add=True)

            kg = k0 - D1
            bg = (b - D1) % NBUF

            @pl.when((kg >= 0) & (kg < NSS))
            def _():
                pltpu.make_async_copy(
                    src2d_hbm.at[pl.ds(0, SS), :], idx_s.at[bg],
                    semE.at[bg]).wait()
                pltpu.make_async_copy(
                    dst2d_hbm.at[pl.ds(0, SS), :], idx_d.at[bg],
                    semE.at[bg]).wait()
                for i in range(SS):
                    pltpu.async_copy(
                        p1_hbm.at[idx_s.at[bg, i]],
                        vals.at[bg, pl.ds(i * CH, CH)], semG.at[bg])

            @pl.when(k0 < NSS)
            def _():
                @pl.when(k0 >= NBUF)
                def _():
                    pltpu.make_async_copy(
                        p1_hbm.at[pl.ds(0, SS * CH)], vals.at[b],
                        semS.at[b]).wait()

                row = rbase + k0 * SS
                pltpu.async_copy(
                    src2d_hbm.at[pl.ds(row, SS), :], idx_s.at[b], semE.at[b])
                pltpu.async_copy(
                    dst2d_hbm.at[pl.ds(row, SS), :], idx_d.at[b], semE.at[b])

        return carry

    lax.fori_loop(0, groups, group, 0)
    for b in range(NBUF):
        last = NSS - NBUF + b

        @pl.when((last >= 0) & (last < NSS))
        def _():
            pltpu.make_async_copy(
                p1_hbm.at[pl.ds(0, SS * CH)], vals.at[b], semS.at[b]).wait()

    plsc.subcore_barrier()
    _flush_1d(acc, out0_hbm, out1_hbm, c, s)


@functools.partial(
    pl.kernel,
    out_type=[jax.ShapeDtypeStruct((NPAD, HH), _f32)] * 2,
    mesh=_mesh,
    compiler_params=_noscale,
    scratch_types=[
        pltpu.VMEM((NBUF, SSC, CH), _i32),
        pltpu.VMEM((NBUF, SSC, CH), _i32),
        pltpu.VMEM((NBUF, SSC * CH, HH), _f32),
        pltpu.VMEM((RZ, HH), _f32),
        pltpu.VMEM_SHARED((NPAD, HH), _f32),
        pltpu.SemaphoreType.DMA((NBUF,)),
        pltpu.SemaphoreType.DMA((NBUF,)),
        pltpu.SemaphoreType.DMA((NBUF,)),
    ],
)
def _sc_agg2(src2d_hbm, dst2d_hbm, p2_hbm, out0_hbm, out1_hbm,
             idx_s, idx_d, rows, zr_v, acc, semE, semG, semS):
    c = lax.axis_index("c")
    s = lax.axis_index("s")
    z16 = jnp.zeros((16,), _f32)
    for i in range(RZ):
        zr_v[i, :] = z16
    for j in range(SP // RZ):
        pltpu.sync_copy(zr_v, acc.at[pl.ds(s * SP + j * RZ, RZ), :])
    plsc.subcore_barrier()
    rbase = (c * NS + s) * NCHUNK
    steps = NSSC + D2
    groups = (steps + NBUF - 1) // NBUF

    def group(g, carry):
        for b in range(NBUF):
            k0 = g * NBUF + b

            ks = k0 - D2
            bs = (b - D2) % NBUF

            @pl.when((ks >= 0) & (ks < NSSC))
            def _():
                pltpu.make_async_copy(
                    p2_hbm.at[pl.ds(0, SSC * CH), :], rows.at[bs],
                    semG.at[bs]).wait()
                for i in range(SSC):
                    pltpu.async_copy(
                        rows.at[bs, pl.ds(i * CH, CH), :],
                        acc.at[idx_d.at[bs, i]], semS.at[bs], add=True)

            kg = k0 - D1
            bg = (b - D1) % NBUF

            @pl.when((kg >= 0) & (kg < NSSC))
            def _():
                pltpu.make_async_copy(
                    src2d_hbm.at[pl.ds(0, SSC), :], idx_s.at[bg],
                    semE.at[bg]).wait()
                pltpu.make_async_copy(
                    dst2d_hbm.at[pl.ds(0, SSC), :], idx_d.at[bg],
                    semE.at[bg]).wait()
                for i in range(SSC):
                    pltpu.async_copy(
                        p2_hbm.at[idx_s.at[bg, i]],
                        rows.at[bg, pl.ds(i * CH, CH), :], semG.at[bg])

            @pl.when(k0 < NSSC)
            def _():
                @pl.when(k0 >= NBUF)
                def _():
                    pltpu.make_async_copy(
                        p2_hbm.at[pl.ds(0, SSC * CH), :], rows.at[b],
                        semS.at[b]).wait()

                row = rbase + k0 * SSC
                pltpu.async_copy(
                    src2d_hbm.at[pl.ds(row, SSC), :], idx_s.at[b], semE.at[b])
                pltpu.async_copy(
                    dst2d_hbm.at[pl.ds(row, SSC), :], idx_d.at[b], semE.at[b])

        return carry

    lax.fori_loop(0, groups, group, 0)
    for b in range(NBUF):
        last = NSSC - NBUF + b

        @pl.when((last >= 0) & (last < NSSC))
        def _():
            pltpu.make_async_copy(
                p2_hbm.at[pl.ds(0, SSC * CH), :], rows.at[b], semS.at[b]).wait()

    plsc.subcore_barrier()

    @pl.when(c == 0)
    def _():
        pltpu.sync_copy(acc.at[pl.ds(s * SP, SP), :],
                        out0_hbm.at[pl.ds(s * SP, SP), :])

    @pl.when(c == 1)
    def _():
        pltpu.sync_copy(acc.at[pl.ds(s * SP, SP), :],
                        out1_hbm.at[pl.ds(s * SP, SP), :])


def _stage0(dega, degb, xp):
    def body(da_r, db_r, x_r, dinv_r, p1_r):
        deg = da_r[:] + db_r[:] + 1.0
        di = lax.rsqrt(deg)
        dinv_r[:] = di
        p1_r[:] = di * x_r[:]

    return pl.pallas_call(
        body,
        out_shape=[jax.ShapeDtypeStruct((NPAD,), _f32),
                   jax.ShapeDtypeStruct((NPAD,), _f32)],
    )(dega, degb, xp)


GB = 49
BR = NPAD // GB


def _stage1(a1a, a1b, dinv, xp, W1, b1, W2):
    def body(aa_r, ab_r, dinv_r, x_r, W1_r, b1_r, W2_r, p2_r):
        di = dinv_r[:]
        s1 = di * (aa_r[:] + ab_r[:]) + di * di * x_r[:]
        h1 = jnp.maximum(s1[:, None] * W1_r[:, :] + b1_r[:, :], 0.0)
        p2_r[:, :] = di[:, None] * jnp.dot(
            h1, W2_r[:, :], preferred_element_type=_f32)

    return pl.pallas_call(
        body,
        grid=(GB,),
        in_specs=[
            pl.BlockSpec((BR,), lambda i: (i,)),
            pl.BlockSpec((BR,), lambda i: (i,)),
            pl.BlockSpec((BR,), lambda i: (i,)),
            pl.BlockSpec((BR,), lambda i: (i,)),
            pl.BlockSpec((1, HH), lambda i: (0, 0)),
            pl.BlockSpec((1, HH), lambda i: (0, 0)),
            pl.BlockSpec((HH, HH), lambda i: (0, 0)),
        ],
        out_specs=pl.BlockSpec((BR, HH), lambda i: (i, 0)),
        out_shape=jax.ShapeDtypeStruct((NPAD, HH), _f32),
    )(a1a, a1b, dinv, xp, W1, b1, W2)


def _stage2(a2a, a2b, p2, dinv, b2, Wl, bl):
    def body(aa_r, ab_r, p2_r, dinv_r, b2_r, Wl_r, bl_r, o_r):
        t = aa_r[:, :] + ab_r[:, :] + p2_r[:, :]
        pre = dinv_r[:][:, None] * t + b2_r[:, :]
        h2 = jnp.maximum(pre, 0.0)
        o_r[:, :] = jnp.dot(h2, Wl_r[:, :],
                            preferred_element_type=_f32) + bl_r[:, :]

    return pl.pallas_call(
        body,
        grid=(GB,),
        in_specs=[
            pl.BlockSpec((BR, HH), lambda i: (i, 0)),
            pl.BlockSpec((BR, HH), lambda i: (i, 0)),
            pl.BlockSpec((BR, HH), lambda i: (i, 0)),
            pl.BlockSpec((BR,), lambda i: (i,)),
            pl.BlockSpec((1, HH), lambda i: (0, 0)),
            pl.BlockSpec((HH, 4), lambda i: (0, 0)),
            pl.BlockSpec((1, 4), lambda i: (0, 0)),
        ],
        out_specs=pl.BlockSpec((BR, 4), lambda i: (i, 0)),
        out_shape=jax.ShapeDtypeStruct((NPAD, 4), _f32),
    )(a2a, a2b, p2, dinv, b2, Wl, bl)


def kernel(x, edges, W1, b1, W2, b2, Wl, bl):
    xp = jnp.pad(x[:, 0], (0, NPAD - NN))
    src2d = edges[0].reshape(EE // CH, CH)
    dst2d = edges[1].reshape(EE // CH, CH)
    dega, degb = _sc_deg(dst2d)
    dinv, p1 = _stage0(dega, degb, xp)
    a1a, a1b = _sc_agg1(src2d, dst2d, p1)
    p2 = _stage1(a1a, a1b, dinv, xp, W1, b1.reshape(1, HH), W2)
    a2a, a2b = _sc_agg2(src2d, dst2d, p2)
    outp = _stage2(a2a, a2b, p2, dinv, b2.reshape(1, HH), Wl,
                   bl.reshape(1, 4))
    return outp[:NN]
